# baseline (speedup 1.0000x reference)
import jax
import jax.numpy as jnp
from jax.experimental import pallas as pl
from jax.experimental.pallas import tpu as pltpu

H = 2048
I = 1024
E = 8
R = 128
TOPK = 2
TM = 512


def _mm(a, b):
    return jax.lax.dot_general(
        a, b, (((1,), (1,)), ((), ())), preferred_element_type=jnp.float32
    )


def _moe_body(xb_ref, lg_ref, bgu_ref, allA_ref, gB_ref, uB_ref, dA_ref,
              bd_ref, dBcat_ref, out_ref, logits_ref):
    logits = lg_ref[...]
    logits_ref[...] = logits

    m = jnp.max(logits, axis=1, keepdims=True)
    p = jnp.exp(logits - m)
    p = p / jnp.sum(p, axis=1, keepdims=True)
    iot = jax.lax.broadcasted_iota(jnp.int32, (TM, E), 1)
    m1 = jnp.max(p, axis=1, keepdims=True)
    a1 = jnp.min(jnp.where(p == m1, iot, E), axis=1, keepdims=True)
    mask1 = iot == a1
    p2 = jnp.where(mask1, -jnp.inf, p)
    m2 = jnp.max(p2, axis=1, keepdims=True)
    a2 = jnp.min(jnp.where(p2 == m2, iot, E), axis=1, keepdims=True)
    wdense = jnp.where(mask1 | (iot == a2), p, 0.0)

    bf = jnp.bfloat16
    xb = xb_ref[...]
    GU = _mm(xb, bgu_ref[...])
    AA = _mm(xb, allA_ref[...]).astype(bf)
    G0 = GU[:, :I]
    U0 = GU[:, I:]

    Hw = jnp.zeros((TM, I), jnp.float32)
    wads = []
    for e in range(E):
        Ag = AA[:, e * 2 * R:e * 2 * R + R]
        Au = AA[:, e * 2 * R + R:(e + 1) * 2 * R]
        g = G0 + _mm(Ag, gB_ref[e])
        u = U0 + _mm(Au, uB_ref[e])
        h = g * jax.nn.sigmoid(g) * u
        we = wdense[:, e:e + 1]
        Hw = Hw + h * we
        Ad = _mm(h.astype(bf), dA_ref[e])
        wads.append((Ad * we).astype(bf))
    WAd = jnp.concatenate(wads, axis=1)
    d = _mm(Hw.astype(bf), bd_ref[...]) + _mm(WAd, dBcat_ref[...])
    out_ref[...] = d


def kernel(hidden_states, gate_w, base_gate, base_up, base_down,
           gate_A, gate_B, up_A, up_B, down_A, down_B):
    b, s_len, h = hidden_states.shape
    T = b * s_len
    x = hidden_states.reshape(T, h)
    router_logits = x @ gate_w.T
    bf = jnp.bfloat16
    bgu = jnp.concatenate([base_gate, base_up], axis=0).astype(bf)
    allA = jnp.concatenate([gate_A, up_A], axis=1).reshape(2 * R * E, H).astype(bf)
    dBcat = jnp.transpose(down_B, (1, 0, 2)).reshape(H, E * R).astype(bf)
    full = lambda shape: pl.BlockSpec(shape, lambda i: (0,) * len(shape))
    final, logits = pl.pallas_call(
        _moe_body,
        grid=(T // TM,),
        in_specs=[
            pl.BlockSpec((TM, H), lambda i: (i, 0)),
            pl.BlockSpec((TM, E), lambda i: (i, 0)),
            full((2 * I, H)),
            full((2 * R * E, H)),
            full((E, I, R)),
            full((E, I, R)),
            full((E, R, I)),
            full((H, I)),
            full((H, E * R)),
        ],
        out_specs=[
            pl.BlockSpec((TM, H), lambda i: (i, 0)),
            pl.BlockSpec((TM, E), lambda i: (i, 0)),
        ],
        out_shape=[
            jax.ShapeDtypeStruct((T, H), jnp.float32),
            jax.ShapeDtypeStruct((T, E), jnp.float32),
        ],
        compiler_params=pltpu.CompilerParams(
            dimension_semantics=("arbitrary",),
        ),
    )(x.astype(bf), router_logits, bgu, allA, gate_B.astype(bf),
      up_B.astype(bf), down_A.astype(bf), base_down.astype(bf), dBcat)
    return final.reshape(b, s_len, h), logits


# speedup vs baseline: 1.0014x; 1.0014x over previous
import jax
import jax.numpy as jnp
from jax.experimental import pallas as pl
from jax.experimental.pallas import tpu as pltpu

H = 2048
I = 1024
E = 8
R = 128
TOPK = 2
TM = 256


def _mm(a, b):
    return jax.lax.dot_general(
        a, b, (((1,), (1,)), ((), ())), preferred_element_type=jnp.float32
    )


def _moe_body(xb_ref, lg_ref, bgu_ref, allA_ref, gB_ref, uB_ref, dA_ref,
              bd_ref, dBcat_ref, out_ref, logits_ref):
    logits = lg_ref[...]
    logits_ref[...] = logits

    m = jnp.max(logits, axis=1, keepdims=True)
    p = jnp.exp(logits - m)
    p = p / jnp.sum(p, axis=1, keepdims=True)
    iot = jax.lax.broadcasted_iota(jnp.int32, (TM, E), 1)
    m1 = jnp.max(p, axis=1, keepdims=True)
    a1 = jnp.min(jnp.where(p == m1, iot, E), axis=1, keepdims=True)
    mask1 = iot == a1
    p2 = jnp.where(mask1, -jnp.inf, p)
    m2 = jnp.max(p2, axis=1, keepdims=True)
    a2 = jnp.min(jnp.where(p2 == m2, iot, E), axis=1, keepdims=True)
    wdense = jnp.where(mask1 | (iot == a2), p, 0.0)

    bf = jnp.bfloat16
    xb = xb_ref[...]
    GU = _mm(xb, bgu_ref[...])
    AA = _mm(xb, allA_ref[...]).astype(bf)
    G0 = GU[:, :I]
    U0 = GU[:, I:]

    Hw = jnp.zeros((TM, I), jnp.float32)
    wads = []
    for e in range(E):
        Ag = AA[:, e * 2 * R:e * 2 * R + R]
        Au = AA[:, e * 2 * R + R:(e + 1) * 2 * R]
        g = G0 + _mm(Ag, gB_ref[e])
        u = U0 + _mm(Au, uB_ref[e])
        h = g * jax.nn.sigmoid(g) * u
        we = wdense[:, e:e + 1]
        Hw = Hw + h * we
        Ad = _mm(h.astype(bf), dA_ref[e])
        wads.append((Ad * we).astype(bf))
    WAd = jnp.concatenate(wads, axis=1)
    d = _mm(Hw.astype(bf), bd_ref[...]) + _mm(WAd, dBcat_ref[...])
    out_ref[...] = d


def kernel(hidden_states, gate_w, base_gate, base_up, base_down,
           gate_A, gate_B, up_A, up_B, down_A, down_B):
    b, s_len, h = hidden_states.shape
    T = b * s_len
    x = hidden_states.reshape(T, h)
    router_logits = x @ gate_w.T
    bf = jnp.bfloat16
    bgu = jnp.concatenate([base_gate, base_up], axis=0).astype(bf)
    allA = jnp.concatenate([gate_A, up_A], axis=1).reshape(2 * R * E, H).astype(bf)
    dBcat = jnp.transpose(down_B, (1, 0, 2)).reshape(H, E * R).astype(bf)
    full = lambda shape: pl.BlockSpec(shape, lambda i: (0,) * len(shape))
    final, logits = pl.pallas_call(
        _moe_body,
        grid=(T // TM,),
        in_specs=[
            pl.BlockSpec((TM, H), lambda i: (i, 0)),
            pl.BlockSpec((TM, E), lambda i: (i, 0)),
            full((2 * I, H)),
            full((2 * R * E, H)),
            full((E, I, R)),
            full((E, I, R)),
            full((E, R, I)),
            full((H, I)),
            full((H, E * R)),
        ],
        out_specs=[
            pl.BlockSpec((TM, H), lambda i: (i, 0)),
            pl.BlockSpec((TM, E), lambda i: (i, 0)),
        ],
        out_shape=[
            jax.ShapeDtypeStruct((T, H), jnp.float32),
            jax.ShapeDtypeStruct((T, E), jnp.float32),
        ],
        compiler_params=pltpu.CompilerParams(
            dimension_semantics=("arbitrary",),
        ),
    )(x.astype(bf), router_logits, bgu, allA, gate_B.astype(bf),
      up_B.astype(bf), down_A.astype(bf), base_down.astype(bf), dBcat)
    return final.reshape(b, s_len, h), logits


# speedup vs baseline: 1.0015x; 1.0001x over previous
import jax
import jax.numpy as jnp
from jax.experimental import pallas as pl
from jax.experimental.pallas import tpu as pltpu

H = 2048
I = 1024
E = 8
R = 128
TOPK = 2
TM = 256


def _mm(a, b):
    return jax.lax.dot_general(
        a, b, (((1,), (1,)), ((), ())), preferred_element_type=jnp.float32
    )


def _moe_body(xb_ref, lg_ref, bgu_ref, allA_ref, gB_ref, uB_ref, dA_ref,
              bd_ref, dBcat_ref, out_ref, logits_ref):
    logits = lg_ref[...]
    logits_ref[...] = logits

    m = jnp.max(logits, axis=1, keepdims=True)
    p = jnp.exp(logits - m)
    p = p / jnp.sum(p, axis=1, keepdims=True)
    iot = jax.lax.broadcasted_iota(jnp.int32, (TM, E), 1)
    m1 = jnp.max(p, axis=1, keepdims=True)
    a1 = jnp.min(jnp.where(p == m1, iot, E), axis=1, keepdims=True)
    mask1 = iot == a1
    p2 = jnp.where(mask1, -jnp.inf, p)
    m2 = jnp.max(p2, axis=1, keepdims=True)
    a2 = jnp.min(jnp.where(p2 == m2, iot, E), axis=1, keepdims=True)
    wdense = jnp.where(mask1 | (iot == a2), p, 0.0)

    bf = jnp.bfloat16
    xb = xb_ref[...]
    GU = _mm(xb, bgu_ref[...])
    AA = _mm(xb, allA_ref[...])
    G0 = GU[:, :I]
    U0 = GU[:, I:]

    Hw = jnp.zeros((TM, I), jnp.float32)
    wads = []
    for e in range(E):
        Ag = AA[:, e * 2 * R:e * 2 * R + R].astype(bf)
        Au = AA[:, e * 2 * R + R:(e + 1) * 2 * R].astype(bf)
        g = G0 + _mm(Ag, gB_ref[e])
        u = U0 + _mm(Au, uB_ref[e])
        h = g * jax.nn.sigmoid(g) * u
        we = wdense[:, e:e + 1]
        Hw = Hw + h * we
        Ad = _mm(h.astype(bf), dA_ref[e])
        wads.append(Ad * we)
    WAd = jnp.concatenate(wads, axis=1).astype(bf)
    d = _mm(Hw.astype(bf), bd_ref[...]) + _mm(WAd, dBcat_ref[...])
    out_ref[...] = d


def kernel(hidden_states, gate_w, base_gate, base_up, base_down,
           gate_A, gate_B, up_A, up_B, down_A, down_B):
    b, s_len, h = hidden_states.shape
    T = b * s_len
    x = hidden_states.reshape(T, h)
    router_logits = x @ gate_w.T
    bf = jnp.bfloat16
    bgu = jnp.concatenate([base_gate, base_up], axis=0).astype(bf)
    allA = jnp.concatenate([gate_A, up_A], axis=1).reshape(2 * R * E, H).astype(bf)
    dBcat = jnp.transpose(down_B, (1, 0, 2)).reshape(H, E * R).astype(bf)
    full = lambda shape: pl.BlockSpec(shape, lambda i: (0,) * len(shape))
    final, logits = pl.pallas_call(
        _moe_body,
        grid=(T // TM,),
        in_specs=[
            pl.BlockSpec((TM, H), lambda i: (i, 0)),
            pl.BlockSpec((TM, E), lambda i: (i, 0)),
            full((2 * I, H)),
            full((2 * R * E, H)),
            full((E, I, R)),
            full((E, I, R)),
            full((E, R, I)),
            full((H, I)),
            full((H, E * R)),
        ],
        out_specs=[
            pl.BlockSpec((TM, H), lambda i: (i, 0)),
            pl.BlockSpec((TM, E), lambda i: (i, 0)),
        ],
        out_shape=[
            jax.ShapeDtypeStruct((T, H), jnp.float32),
            jax.ShapeDtypeStruct((T, E), jnp.float32),
        ],
        compiler_params=pltpu.CompilerParams(
            dimension_semantics=("arbitrary",),
        ),
    )(x.astype(bf), router_logits, bgu, allA, gate_B.astype(bf),
      up_B.astype(bf), down_A.astype(bf), base_down.astype(bf), dBcat)
    return final.reshape(b, s_len, h), logits


# speedup vs baseline: 1.0550x; 1.0535x over previous
import jax
import jax.numpy as jnp
from jax.experimental import pallas as pl
from jax.experimental.pallas import tpu as pltpu

H = 2048
I = 1024
E = 8
R = 128
TOPK = 2
TM = 256


def _mm(a, b):
    return jax.lax.dot_general(
        a, b, (((1,), (1,)), ((), ())), preferred_element_type=jnp.float32
    )


def _moe_body(xb_ref, lg_ref, bgu_ref, allA_ref, gB_ref, uB_ref, dA_ref,
              bd_ref, dBcat_ref, out_ref, logits_ref):
    logits = lg_ref[...]
    logits_ref[...] = logits

    m = jnp.max(logits, axis=1, keepdims=True)
    p = jnp.exp(logits - m)
    p = p / jnp.sum(p, axis=1, keepdims=True)
    iot = jax.lax.broadcasted_iota(jnp.int32, (TM, E), 1)
    m1 = jnp.max(p, axis=1, keepdims=True)
    a1 = jnp.min(jnp.where(p == m1, iot, E), axis=1, keepdims=True)
    mask1 = iot == a1
    p2 = jnp.where(mask1, -jnp.inf, p)
    m2 = jnp.max(p2, axis=1, keepdims=True)
    a2 = jnp.min(jnp.where(p2 == m2, iot, E), axis=1, keepdims=True)
    wdense = jnp.where(mask1 | (iot == a2), p, 0.0)

    bf = jnp.bfloat16
    xb = xb_ref[...].astype(bf)
    GU = _mm(xb, bgu_ref[...])
    AA = _mm(xb, allA_ref[...])
    G0 = GU[:, :I]
    U0 = GU[:, I:]

    Hw = jnp.zeros((TM, I), jnp.float32)
    wads = []
    for e in range(E):
        Ag = AA[:, e * 2 * R:e * 2 * R + R].astype(bf)
        Au = AA[:, e * 2 * R + R:(e + 1) * 2 * R].astype(bf)
        g = G0 + _mm(Ag, gB_ref[e])
        u = U0 + _mm(Au, uB_ref[e])
        h = g * jax.nn.sigmoid(g) * u
        we = wdense[:, e:e + 1]
        Hw = Hw + h * we
        Ad = _mm(h.astype(bf), dA_ref[e])
        wads.append(Ad * we)
    WAd = jnp.concatenate(wads, axis=1).astype(bf)
    d = _mm(Hw.astype(bf), bd_ref[...]) + _mm(WAd, dBcat_ref[...])
    out_ref[...] = d


def kernel(hidden_states, gate_w, base_gate, base_up, base_down,
           gate_A, gate_B, up_A, up_B, down_A, down_B):
    b, s_len, h = hidden_states.shape
    T = b * s_len
    x = hidden_states.reshape(T, h)
    router_logits = x @ gate_w.T
    bf = jnp.bfloat16
    bgu = jnp.concatenate([base_gate, base_up], axis=0).astype(bf)
    allA = jnp.concatenate([gate_A, up_A], axis=1).reshape(2 * R * E, H).astype(bf)
    dBcat = jnp.transpose(down_B, (1, 0, 2)).reshape(H, E * R).astype(bf)
    full = lambda shape: pl.BlockSpec(shape, lambda i: (0,) * len(shape))
    final, logits = pl.pallas_call(
        _moe_body,
        grid=(T // TM,),
        in_specs=[
            pl.BlockSpec((TM, H), lambda i: (i, 0)),
            pl.BlockSpec((TM, E), lambda i: (i, 0)),
            full((2 * I, H)),
            full((2 * R * E, H)),
            full((E, I, R)),
            full((E, I, R)),
            full((E, R, I)),
            full((H, I)),
            full((H, E * R)),
        ],
        out_specs=[
            pl.BlockSpec((TM, H), lambda i: (i, 0)),
            pl.BlockSpec((TM, E), lambda i: (i, 0)),
        ],
        out_shape=[
            jax.ShapeDtypeStruct((T, H), jnp.float32),
            jax.ShapeDtypeStruct((T, E), jnp.float32),
        ],
        compiler_params=pltpu.CompilerParams(
            dimension_semantics=("arbitrary",),
        ),
    )(x, router_logits, bgu, allA, gate_B.astype(bf),
      up_B.astype(bf), down_A.astype(bf), base_down.astype(bf), dBcat)
    return final.reshape(b, s_len, h), logits
